# baseline (speedup 1.0000x reference)
import functools

import jax
import jax.numpy as jnp
from jax import lax
from jax.experimental import pallas as pl
from jax.experimental.pallas import tpu as pltpu
from jax.experimental.pallas import tpu_sc as plsc

N_DOCS = 100000
M = 16
K = 256
SUB = 8
D = M * SUB
BATCH = 1024
NEG_TOP_K = 200

NB = 2048
GRID1 = (N_DOCS + NB - 1) // NB
P = GRID1 * NB
NEG_INF = -1e30


NIDX = P * M
NW = 32
PER_W = NIDX // NW
SC_CHUNK = 3136
SC_NCH = PER_W // SC_CHUNK


def _sc_decode_body(table_hbm, idx_hbm, out_hbm,
                    idx_v, rows0, rows1, sem_g0, sem_g1, sem_o0, sem_o1):
    wid = lax.axis_index("s") * 2 + lax.axis_index("c")
    base = wid * PER_W
    pltpu.sync_copy(idx_hbm.at[pl.ds(base, PER_W)], idx_v)
    rows = (rows0, rows1)
    sem_g = (sem_g0, sem_g1)
    sem_o = (sem_o0, sem_o1)
    out_h = [None, None]
    for c in range(SC_NCH):
        b = c % 2
        if out_h[b] is not None:
            out_h[b].wait()
        pltpu.async_copy(
            table_hbm.at[idx_v.at[pl.ds(c * SC_CHUNK, SC_CHUNK)]],
            rows[b], sem_g[b]).wait()
        out_h[b] = pltpu.async_copy(
            rows[b], out_hbm.at[pl.ds(base + c * SC_CHUNK, SC_CHUNK)],
            sem_o[b])
    out_h[0].wait()
    out_h[1].wait()


def _sc_decode(centroids, codes_pad):
    flat_idx = (codes_pad + (jnp.arange(M, dtype=jnp.int32) * K)[None, :]
                ).reshape(NIDX)
    table = centroids.reshape(M * K, SUB)
    mesh = plsc.VectorSubcoreMesh(core_axis_name="c", subcore_axis_name="s")
    dec = pl.kernel(
        _sc_decode_body,
        out_type=jax.ShapeDtypeStruct((NIDX, SUB), jnp.float32),
        mesh=mesh,
        scratch_types=[
            pltpu.VMEM((PER_W,), jnp.int32),
            pltpu.VMEM((SC_CHUNK, SUB), jnp.float32),
            pltpu.VMEM((SC_CHUNK, SUB), jnp.float32),
            pltpu.SemaphoreType.DMA,
            pltpu.SemaphoreType.DMA,
            pltpu.SemaphoreType.DMA,
            pltpu.SemaphoreType.DMA,
        ],
        compiler_params=pltpu.CompilerParams(use_tc_tiling_on_sc=False),
    )
    return dec(table, flat_idx).reshape(P, D)


def _scores_body(q_ref, e_ref, pos_ref,
                 s_ref, mx_ref, mn_ref, su_ref, sq_ref, rp_ref):
    j = pl.program_id(0)
    s = jax.lax.dot_general(q_ref[...], e_ref[...],
                            (((1,), (1,)), ((), ())),
                            preferred_element_type=jnp.float32)
    col = j * NB + jax.lax.broadcasted_iota(jnp.int32, (1, NB), 1)
    real = col < N_DOCS
    s = jnp.where(real, s, NEG_INF)
    s_ref[...] = s
    mx_ref[...] = jnp.max(s, axis=1, keepdims=True)[None]
    mn_ref[...] = jnp.min(jnp.where(real, s, 1e30), axis=1, keepdims=True)[None]
    sr = jnp.where(real, s, 0.0)
    su_ref[...] = jnp.sum(sr, axis=1, keepdims=True)[None]
    sq_ref[...] = jnp.sum(sr * sr, axis=1, keepdims=True)[None]
    pos = pos_ref[0, :]
    rp_ref[...] = jnp.sum(
        jnp.where(col == pos[:, None], s, 0.0), axis=1, keepdims=True)[None]


def _compute_scores(query_embeds, centroids, codes, pos_pids):
    codes32 = codes.astype(jnp.int32)
    codes_pad = jnp.pad(codes32, ((0, P - N_DOCS), (0, 0)))
    e = _sc_decode(centroids, codes_pad)
    pos2d = pos_pids.astype(jnp.int32).reshape(1, BATCH)
    part = jax.ShapeDtypeStruct((GRID1, BATCH, 1), jnp.float32)
    return pl.pallas_call(
        _scores_body,
        grid=(GRID1,),
        in_specs=[
            pl.BlockSpec((BATCH, D), lambda j: (0, 0)),
            pl.BlockSpec((NB, D), lambda j: (j, 0)),
            pl.BlockSpec((1, BATCH), lambda j: (0, 0)),
        ],
        out_specs=[
            pl.BlockSpec((BATCH, NB), lambda j: (0, j)),
            pl.BlockSpec((1, BATCH, 1), lambda j: (j, 0, 0)),
            pl.BlockSpec((1, BATCH, 1), lambda j: (j, 0, 0)),
            pl.BlockSpec((1, BATCH, 1), lambda j: (j, 0, 0)),
            pl.BlockSpec((1, BATCH, 1), lambda j: (j, 0, 0)),
            pl.BlockSpec((1, BATCH, 1), lambda j: (j, 0, 0)),
        ],
        out_shape=[jax.ShapeDtypeStruct((BATCH, P), jnp.float32),
                   part, part, part, part, part],
    )(query_embeds, e, pos2d)


QB = 32
NQ = BATCH // QB
SEL_ITERS = 4
NT = 4
NLAD = 8


def _loss_body(mx_ref, mn_ref, su_ref, sq_ref, rp_ref, s_ref, out_ref):
    g = pl.program_id(0)
    s = s_ref[...]
    m = jnp.max(mx_ref[...], axis=0)
    mn = jnp.min(mn_ref[...], axis=0)
    rel = jnp.sum(rp_ref[...], axis=0)
    mu = jnp.sum(su_ref[...], axis=0) / N_DOCS
    var = jnp.sum(sq_ref[...], axis=0) / N_DOCS - mu * mu
    sd = jnp.sqrt(jnp.maximum(var, 0.0))

    def refine(carry, thresholds):
        lo, hi = carry
        for t in thresholds:
            c = jnp.sum((s >= t).astype(jnp.float32), axis=1, keepdims=True)
            ge = c >= NEG_TOP_K
            lo = jnp.where(ge, jnp.maximum(lo, t), lo)
            hi = jnp.where(ge, hi, jnp.minimum(hi, t))
        return lo, hi

    ladder = [mu + sd * (2.0 + 0.4 * j) for j in range(NLAD)]
    lo, hi = refine((mn, m + 1.0), ladder)

    def body(_, carry):
        lo, hi = carry
        step = (hi - lo) / (NT + 1)
        return refine((lo, hi), [lo + step * (j + 1) for j in range(NT)])

    lo, hi = jax.lax.fori_loop(0, SEL_ITERS, body, (lo, hi))

    ex = jnp.exp(s - m)
    ge_hi = s >= hi
    c_hi = jnp.sum(ge_hi.astype(jnp.float32), axis=1, keepdims=True)
    sum_hi = jnp.sum(jnp.where(ge_hi, ex, 0.0), axis=1, keepdims=True)
    total = sum_hi + (NEG_TOP_K - c_hi) * jnp.exp(0.5 * (lo + hi) - m)
    row_loss = jnp.log(jnp.exp(rel - m) + total) + m - rel

    @pl.when(g == 0)
    def _():
        out_ref[...] = jnp.zeros_like(out_ref)

    out_ref[...] += (jnp.sum(row_loss) / BATCH).reshape(1, 1)


def kernel(query_embeds, centroids, codes, pos_pids):
    s, mx, mn, su, sq, rp = _compute_scores(
        query_embeds, centroids, codes, pos_pids)
    stat_spec = pl.BlockSpec((GRID1, QB, 1), lambda g: (0, g, 0))
    out = pl.pallas_call(
        _loss_body,
        grid=(NQ,),
        in_specs=[stat_spec, stat_spec, stat_spec, stat_spec, stat_spec,
                  pl.BlockSpec((QB, P), lambda g: (g, 0))],
        out_specs=pl.BlockSpec((1, 1), lambda g: (0, 0)),
        out_shape=jax.ShapeDtypeStruct((1, 1), jnp.float32),
    )(mx, mn, su, sq, rp, s)
    return out[0, 0]


# speedup vs baseline: 1.1034x; 1.1034x over previous
import functools

import jax
import jax.numpy as jnp
from jax import lax
from jax.experimental import pallas as pl
from jax.experimental.pallas import tpu as pltpu
from jax.experimental.pallas import tpu_sc as plsc

N_DOCS = 100000
M = 16
K = 256
SUB = 8
D = M * SUB
BATCH = 1024
NEG_TOP_K = 200

NB = 2048
GRID1 = (N_DOCS + NB - 1) // NB
P = GRID1 * NB
NEG_INF = -1e30


NIDX = P * M
NW = 32
PER_W = NIDX // NW
SC_CHUNK = 3136
SC_NCH = PER_W // SC_CHUNK


def _sc_decode_body(table_hbm, idx_hbm, out_hbm,
                    idx_v, rows0, rows1, sem_g0, sem_g1, sem_o0, sem_o1):
    wid = lax.axis_index("s") * 2 + lax.axis_index("c")
    base = wid * PER_W
    pltpu.sync_copy(idx_hbm.at[pl.ds(base, PER_W)], idx_v)
    rows = (rows0, rows1)
    sem_g = (sem_g0, sem_g1)
    sem_o = (sem_o0, sem_o1)
    out_h = [None, None]
    for c in range(SC_NCH):
        b = c % 2
        if out_h[b] is not None:
            out_h[b].wait()
        pltpu.async_copy(
            table_hbm.at[idx_v.at[pl.ds(c * SC_CHUNK, SC_CHUNK)]],
            rows[b], sem_g[b]).wait()
        out_h[b] = pltpu.async_copy(
            rows[b], out_hbm.at[pl.ds(base + c * SC_CHUNK, SC_CHUNK)],
            sem_o[b])
    out_h[0].wait()
    out_h[1].wait()


def _sc_decode(centroids, codes_pad):
    flat_idx = (codes_pad + (jnp.arange(M, dtype=jnp.int32) * K)[None, :]
                ).reshape(NIDX)
    table = centroids.reshape(M * K, SUB)
    mesh = plsc.VectorSubcoreMesh(core_axis_name="c", subcore_axis_name="s")
    dec = pl.kernel(
        _sc_decode_body,
        out_type=jax.ShapeDtypeStruct((NIDX, SUB), jnp.float32),
        mesh=mesh,
        scratch_types=[
            pltpu.VMEM((PER_W,), jnp.int32),
            pltpu.VMEM((SC_CHUNK, SUB), jnp.float32),
            pltpu.VMEM((SC_CHUNK, SUB), jnp.float32),
            pltpu.SemaphoreType.DMA,
            pltpu.SemaphoreType.DMA,
            pltpu.SemaphoreType.DMA,
            pltpu.SemaphoreType.DMA,
        ],
        compiler_params=pltpu.CompilerParams(use_tc_tiling_on_sc=False),
    )
    return dec(table, flat_idx).reshape(P, D)


def _scores_body(q_ref, e_ref, pos_ref,
                 s_ref, mx_ref, mn_ref, rp_ref):
    j = pl.program_id(0)
    s = jax.lax.dot_general(q_ref[...], e_ref[...],
                            (((1,), (1,)), ((), ())),
                            preferred_element_type=jnp.float32)
    col = j * NB + jax.lax.broadcasted_iota(jnp.int32, (1, NB), 1)
    real = col < N_DOCS
    s = jnp.where(real, s, NEG_INF)
    s_ref[...] = s
    mx_ref[...] = jnp.max(s, axis=1, keepdims=True)[None]
    mn_ref[...] = jnp.min(jnp.where(real, s, 1e30), axis=1, keepdims=True)[None]
    pos = pos_ref[0, :]
    rp_ref[...] = jnp.sum(
        jnp.where(col == pos[:, None], s, 0.0), axis=1, keepdims=True)[None]


def _compute_scores(query_embeds, centroids, codes, pos_pids):
    codes32 = codes.astype(jnp.int32)
    codes_pad = jnp.pad(codes32, ((0, P - N_DOCS), (0, 0)))
    e = _sc_decode(centroids, codes_pad)
    pos2d = pos_pids.astype(jnp.int32).reshape(1, BATCH)
    part = jax.ShapeDtypeStruct((GRID1, BATCH, 1), jnp.float32)
    return pl.pallas_call(
        _scores_body,
        grid=(GRID1,),
        in_specs=[
            pl.BlockSpec((BATCH, D), lambda j: (0, 0)),
            pl.BlockSpec((NB, D), lambda j: (j, 0)),
            pl.BlockSpec((1, BATCH), lambda j: (0, 0)),
        ],
        out_specs=[
            pl.BlockSpec((BATCH, NB), lambda j: (0, j)),
            pl.BlockSpec((1, BATCH, 1), lambda j: (j, 0, 0)),
            pl.BlockSpec((1, BATCH, 1), lambda j: (j, 0, 0)),
            pl.BlockSpec((1, BATCH, 1), lambda j: (j, 0, 0)),
        ],
        out_shape=[jax.ShapeDtypeStruct((BATCH, P), jnp.float32),
                   part, part, part],
    )(query_embeds, e, pos2d)


QB = 32
NQ = BATCH // QB
SEL_ITERS = 16


def _loss_body(mx_ref, mn_ref, rp_ref, s_ref, out_ref):
    g = pl.program_id(0)
    s = s_ref[...]
    m = jnp.max(mx_ref[...], axis=0)
    mn = jnp.min(mn_ref[...], axis=0)
    rel = jnp.sum(rp_ref[...], axis=0)

    def body(_, carry):
        lo, hi = carry
        mid = 0.5 * (lo + hi)
        c = jnp.sum((s >= mid).astype(jnp.float32), axis=1, keepdims=True)
        ge = c >= NEG_TOP_K
        return jnp.where(ge, mid, lo), jnp.where(ge, hi, mid)

    lo, hi = jax.lax.fori_loop(0, SEL_ITERS, body, (mn, m + 1.0))

    ex = jnp.exp(s - m)
    ge_hi = s >= hi
    c_hi = jnp.sum(ge_hi.astype(jnp.float32), axis=1, keepdims=True)
    sum_hi = jnp.sum(jnp.where(ge_hi, ex, 0.0), axis=1, keepdims=True)
    total = sum_hi + (NEG_TOP_K - c_hi) * jnp.exp(0.5 * (lo + hi) - m)
    row_loss = jnp.log(jnp.exp(rel - m) + total) + m - rel

    @pl.when(g == 0)
    def _():
        out_ref[...] = jnp.zeros_like(out_ref)

    out_ref[...] += (jnp.sum(row_loss) / BATCH).reshape(1, 1)


def kernel(query_embeds, centroids, codes, pos_pids):
    s, mx, mn, rp = _compute_scores(
        query_embeds, centroids, codes, pos_pids)
    stat_spec = pl.BlockSpec((GRID1, QB, 1), lambda g: (0, g, 0))
    out = pl.pallas_call(
        _loss_body,
        grid=(NQ,),
        in_specs=[stat_spec, stat_spec, stat_spec,
                  pl.BlockSpec((QB, P), lambda g: (g, 0))],
        out_specs=pl.BlockSpec((1, 1), lambda g: (0, 0)),
        out_shape=jax.ShapeDtypeStruct((1, 1), jnp.float32),
    )(mx, mn, rp, s)
    return out[0, 0]


# speedup vs baseline: 1.1417x; 1.0348x over previous
import functools

import jax
import jax.numpy as jnp
from jax import lax
from jax.experimental import pallas as pl
from jax.experimental.pallas import tpu as pltpu
from jax.experimental.pallas import tpu_sc as plsc

N_DOCS = 100000
M = 16
K = 256
SUB = 8
D = M * SUB
BATCH = 1024
NEG_TOP_K = 200

NB = 2048
GRID1 = (N_DOCS + NB - 1) // NB
P = GRID1 * NB
NEG_INF = -1e30

HALF_BLOCKS = (25, 24)
HALF_DOCS = tuple(b * NB for b in HALF_BLOCKS)
HALF_BASE = (0, HALF_DOCS[0])

NW = 32


def _make_sc_body(per_w, chunk):
    nch = per_w // chunk

    def body(table_hbm, idx_hbm, out_hbm,
             idx_v, rows0, rows1, sem_g0, sem_g1, sem_o0, sem_o1):
        wid = lax.axis_index("s") * 2 + lax.axis_index("c")
        base = wid * per_w
        pltpu.sync_copy(idx_hbm.at[pl.ds(base, per_w)], idx_v)
        rows = (rows0, rows1)
        sem_g = (sem_g0, sem_g1)
        sem_o = (sem_o0, sem_o1)
        out_h = [None, None]
        for c in range(nch):
            b = c % 2
            if out_h[b] is not None:
                out_h[b].wait()
            pltpu.async_copy(
                table_hbm.at[idx_v.at[pl.ds(c * chunk, chunk)]],
                rows[b], sem_g[b]).wait()
            out_h[b] = pltpu.async_copy(
                rows[b], out_hbm.at[pl.ds(base + c * chunk, chunk)],
                sem_o[b])
        out_h[0].wait()
        out_h[1].wait()

    return body


def _sc_decode(table, flat_idx, chunk):
    nidx = flat_idx.shape[0]
    per_w = nidx // NW
    mesh = plsc.VectorSubcoreMesh(core_axis_name="c", subcore_axis_name="s")
    dec = pl.kernel(
        _make_sc_body(per_w, chunk),
        out_type=jax.ShapeDtypeStruct((nidx, SUB), jnp.float32),
        mesh=mesh,
        scratch_types=[
            pltpu.VMEM((per_w,), jnp.int32),
            pltpu.VMEM((chunk, SUB), jnp.float32),
            pltpu.VMEM((chunk, SUB), jnp.float32),
            pltpu.SemaphoreType.DMA,
            pltpu.SemaphoreType.DMA,
            pltpu.SemaphoreType.DMA,
            pltpu.SemaphoreType.DMA,
        ],
        compiler_params=pltpu.CompilerParams(use_tc_tiling_on_sc=False),
    )
    return dec(table, flat_idx)


def _scores_body(q_ref, e_ref, pos_ref,
                 s_ref, mx_ref, mn_ref, rp_ref, *, base):
    j = pl.program_id(0)
    s = jax.lax.dot_general(q_ref[...], e_ref[...],
                            (((1,), (1,)), ((), ())),
                            preferred_element_type=jnp.float32)
    col = base + j * NB + jax.lax.broadcasted_iota(jnp.int32, (1, NB), 1)
    real = col < N_DOCS
    s = jnp.where(real, s, NEG_INF)
    s_ref[...] = s
    mx_ref[...] = jnp.max(s, axis=1, keepdims=True)[None]
    mn_ref[...] = jnp.min(jnp.where(real, s, 1e30), axis=1, keepdims=True)[None]
    pos = pos_ref[0, :]
    rp_ref[...] = jnp.sum(
        jnp.where(col == pos[:, None], s, 0.0), axis=1, keepdims=True)[None]


def _scores_half(query_embeds, e_half, pos2d, nblocks, base):
    nd = nblocks * NB
    part = jax.ShapeDtypeStruct((nblocks, BATCH, 1), jnp.float32)
    return pl.pallas_call(
        functools.partial(_scores_body, base=base),
        grid=(nblocks,),
        in_specs=[
            pl.BlockSpec((BATCH, D), lambda j: (0, 0)),
            pl.BlockSpec((NB, D), lambda j: (j, 0)),
            pl.BlockSpec((1, BATCH), lambda j: (0, 0)),
        ],
        out_specs=[
            pl.BlockSpec((BATCH, NB), lambda j: (0, j)),
            pl.BlockSpec((1, BATCH, 1), lambda j: (j, 0, 0)),
            pl.BlockSpec((1, BATCH, 1), lambda j: (j, 0, 0)),
            pl.BlockSpec((1, BATCH, 1), lambda j: (j, 0, 0)),
        ],
        out_shape=[jax.ShapeDtypeStruct((BATCH, nd), jnp.float32),
                   part, part, part],
    )(query_embeds, e_half, pos2d)


QB = 32
NQ = BATCH // QB
SEL_ITERS = 16


def _loss_body(mx_ref, mn_ref, rp_ref, sa_ref, sb_ref, out_ref):
    g = pl.program_id(0)
    sa = sa_ref[...]
    sb = sb_ref[...]
    m = jnp.max(mx_ref[...], axis=0)
    mn = jnp.min(mn_ref[...], axis=0)
    rel = jnp.sum(rp_ref[...], axis=0)

    def body(_, carry):
        lo, hi = carry
        mid = 0.5 * (lo + hi)
        c = (jnp.sum((sa >= mid).astype(jnp.float32), axis=1, keepdims=True)
             + jnp.sum((sb >= mid).astype(jnp.float32), axis=1, keepdims=True))
        ge = c >= NEG_TOP_K
        return jnp.where(ge, mid, lo), jnp.where(ge, hi, mid)

    lo, hi = jax.lax.fori_loop(0, SEL_ITERS, body, (mn, m + 1.0))

    total = jnp.zeros((QB, 1), jnp.float32)
    c_hi = jnp.zeros((QB, 1), jnp.float32)
    for s in (sa, sb):
        ex = jnp.exp(s - m)
        ge_hi = s >= hi
        c_hi += jnp.sum(ge_hi.astype(jnp.float32), axis=1, keepdims=True)
        total += jnp.sum(jnp.where(ge_hi, ex, 0.0), axis=1, keepdims=True)
    total += (NEG_TOP_K - c_hi) * jnp.exp(0.5 * (lo + hi) - m)
    row_loss = jnp.log(jnp.exp(rel - m) + total) + m - rel

    @pl.when(g == 0)
    def _():
        out_ref[...] = jnp.zeros_like(out_ref)

    out_ref[...] += (jnp.sum(row_loss) / BATCH).reshape(1, 1)


def kernel(query_embeds, centroids, codes, pos_pids):
    codes32 = codes.astype(jnp.int32)
    codes_pad = jnp.pad(codes32, ((0, P - N_DOCS), (0, 0)))
    flat_idx = (codes_pad + (jnp.arange(M, dtype=jnp.int32) * K)[None, :]
                ).reshape(P * M)
    table = centroids.reshape(M * K, SUB)
    pos2d = pos_pids.astype(jnp.int32).reshape(1, BATCH)

    halves = []
    for h in (0, 1):
        nidx = HALF_DOCS[h] * M
        idx_h = lax.dynamic_slice_in_dim(flat_idx, HALF_BASE[h] * M, nidx)
        chunk = (nidx // NW) // 8
        e_h = _sc_decode(table, idx_h, chunk).reshape(HALF_DOCS[h], D)
        halves.append(_scores_half(query_embeds, e_h, pos2d,
                                   HALF_BLOCKS[h], HALF_BASE[h]))

    (sa, mxa, mna, rpa), (sb, mxb, mnb, rpb) = halves
    mx = jnp.concatenate([mxa, mxb], axis=0)
    mn = jnp.concatenate([mna, mnb], axis=0)
    rp = jnp.concatenate([rpa, rpb], axis=0)

    stat_spec = pl.BlockSpec((GRID1, QB, 1), lambda g: (0, g, 0))
    out = pl.pallas_call(
        _loss_body,
        grid=(NQ,),
        in_specs=[stat_spec, stat_spec, stat_spec,
                  pl.BlockSpec((QB, HALF_DOCS[0]), lambda g: (g, 0)),
                  pl.BlockSpec((QB, HALF_DOCS[1]), lambda g: (g, 0))],
        out_specs=pl.BlockSpec((1, 1), lambda g: (0, 0)),
        out_shape=jax.ShapeDtypeStruct((1, 1), jnp.float32),
    )(mx, mn, rp, sa, sb)
    return out[0, 0]
